# Initial kernel scaffold; baseline (speedup 1.0000x reference)
import functools

import jax
import jax.numpy as jnp
from jax import lax
from jax.experimental import pallas as pl
from jax.experimental.pallas import tpu as pltpu
from jax.experimental.pallas import tpu_sc as plsc

NC, NS, L = 2, 16, 16
NW = NC * NS
CHUNK = 128


def _tr_body(f_ref, o_ref):
    o_ref[...] = jnp.transpose(f_ref[...], (0, 2, 1))


def _transpose(f3):
    b, c, npix = f3.shape
    return pl.pallas_call(
        _tr_body,
        grid=(b,),
        in_specs=[pl.BlockSpec((1, c, npix), lambda i: (i, 0, 0))],
        out_specs=pl.BlockSpec((1, npix, c), lambda i: (i, 0, 0)),
        out_shape=jax.ShapeDtypeStruct((b, npix, c), f3.dtype),
    )(f3)


def _make_sc_gather(B, N, C, W, npix):
    pts_per_w = (B * N) // NW
    w_per_b = N // pts_per_w
    nchunk = pts_per_w // CHUNK
    mesh = plsc.VectorSubcoreMesh(
        core_axis_name="c", subcore_axis_name="s", num_cores=NC, num_subcores=NS
    )

    @functools.partial(
        pl.kernel,
        out_type=jax.ShapeDtypeStruct((B, N, C), jnp.float32),
        mesh=mesh,
        scratch_types=[
            pltpu.VMEM((pts_per_w, 2), jnp.float32),
            pltpu.VMEM((pts_per_w,), jnp.int32),
            pltpu.VMEM((CHUNK, C), jnp.float32),
            pltpu.SemaphoreType.DMA,
        ],
    )
    def sc_gather(loc_hbm, table_hbm, out_hbm, loc_v, idx_v, rows_v, sem):
        cid = lax.axis_index("c")
        sid = lax.axis_index("s")
        wid = sid * NC + cid
        b = wid // w_per_b
        noff = (wid % w_per_b) * pts_per_w

        pltpu.sync_copy(loc_hbm.at[b, pl.ds(noff, pts_per_w), :], loc_v)

        half = jnp.float32(W / 2.0)

        def idx_body(j, carry):
            pids = lax.iota(jnp.int32, L) + j * L
            lane0 = jnp.zeros((L,), jnp.int32)
            x = plsc.load_gather(loc_v, [pids, lane0])
            y = plsc.load_gather(loc_v, [pids, lane0 + 1])
            x = jnp.clip(x, -0.999, 0.999)
            y = jnp.clip(y, -0.999, 0.999)
            row = ((jnp.float32(1.0) - y) * half).astype(jnp.int32)
            col = ((jnp.float32(1.0) + x) * half).astype(jnp.int32)
            pix = row * W + col + b * npix
            idx_v[pl.ds(j * L, L)] = pix
            return carry

        lax.fori_loop(0, pts_per_w // L, idx_body, 0, unroll=4)

        for ch in range(nchunk):
            cp = pltpu.async_copy(
                table_hbm.at[idx_v.at[pl.ds(ch * CHUNK, CHUNK)]], rows_v, sem
            )
            cp.wait()
            pltpu.sync_copy(
                rows_v, out_hbm.at[b, pl.ds(noff + ch * CHUNK, CHUNK), :]
            )

    return sc_gather


def kernel(fmap, loc):
    B, C, H, W = fmap.shape
    N = loc.shape[1]
    npix = H * W
    table = _transpose(fmap.reshape(B, C, npix)).reshape(B * npix, C)
    sc_gather = _make_sc_gather(B, N, C, W, npix)
    return sc_gather(loc, table)


# speedup vs baseline: 5.8592x; 5.8592x over previous
import functools

import jax
import jax.numpy as jnp
from jax import lax
from jax.experimental import pallas as pl
from jax.experimental.pallas import tpu as pltpu
from jax.experimental.pallas import tpu_sc as plsc

NC, NS, L = 2, 16, 16
NW = NC * NS
CHUNK = 128


def _tr_body(f_ref, o_ref):
    o_ref[...] = jnp.transpose(f_ref[...], (0, 2, 1))


def _transpose(f3):
    b, c, npix = f3.shape
    return pl.pallas_call(
        _tr_body,
        grid=(b,),
        in_specs=[pl.BlockSpec((1, c, npix), lambda i: (i, 0, 0))],
        out_specs=pl.BlockSpec((1, npix, c), lambda i: (i, 0, 0)),
        out_shape=jax.ShapeDtypeStruct((b, npix, c), f3.dtype),
    )(f3)


def _make_sc_gather(B, N, C, W, npix):
    pts_per_w = (B * N) // NW
    w_per_b = N // pts_per_w
    nchunk = pts_per_w // CHUNK
    mesh = plsc.VectorSubcoreMesh(
        core_axis_name="c", subcore_axis_name="s", num_cores=NC, num_subcores=NS
    )

    @functools.partial(
        pl.kernel,
        out_type=jax.ShapeDtypeStruct((B, N, C), jnp.float32),
        mesh=mesh,
        compiler_params=pltpu.CompilerParams(needs_layout_passes=False),
        scratch_types=[
            pltpu.VMEM((pts_per_w * 2,), jnp.float32),
            pltpu.VMEM((pts_per_w,), jnp.int32),
            pltpu.VMEM((CHUNK, C), jnp.float32),
            pltpu.SemaphoreType.DMA,
        ],
    )
    def sc_gather(loc_hbm, table_hbm, out_hbm, loc_v, idx_v, rows_v, sem):
        cid = lax.axis_index("c")
        sid = lax.axis_index("s")
        wid = sid * NC + cid
        b = wid // w_per_b
        noff = (wid % w_per_b) * pts_per_w

        pltpu.sync_copy(loc_hbm.at[b, pl.ds(noff * 2, pts_per_w * 2)], loc_v)

        half = jnp.float32(W / 2.0)

        def idx_body(j, carry):
            xpos = lax.iota(jnp.int32, L) * 2 + j * (2 * L)
            x = plsc.load_gather(loc_v, [xpos])
            y = plsc.load_gather(loc_v, [xpos + 1])
            x = jnp.clip(x, -0.999, 0.999)
            y = jnp.clip(y, -0.999, 0.999)
            row = ((jnp.float32(1.0) - y) * half).astype(jnp.int32)
            col = ((jnp.float32(1.0) + x) * half).astype(jnp.int32)
            pix = row * W + col + b * npix
            idx_v[pl.ds(j * L, L)] = pix
            return carry

        lax.fori_loop(0, pts_per_w // L, idx_body, 0, unroll=4)

        for ch in range(nchunk):
            cp = pltpu.async_copy(
                table_hbm.at[idx_v.at[pl.ds(ch * CHUNK, CHUNK)]], rows_v, sem
            )
            cp.wait()
            pltpu.sync_copy(
                rows_v, out_hbm.at[b, pl.ds(noff + ch * CHUNK, CHUNK), :]
            )

    return sc_gather


def kernel(fmap, loc):
    B, C, H, W = fmap.shape
    N = loc.shape[1]
    npix = H * W
    table = _transpose(fmap.reshape(B, C, npix)).reshape(B * npix, C)
    sc_gather = _make_sc_gather(B, N, C, W, npix)
    return sc_gather(loc.reshape(B, N * 2), table)
